# baseline (speedup 1.0000x reference)
import functools

import jax
import jax.numpy as jnp
from jax import lax
from jax.experimental import pallas as pl
from jax.experimental.pallas import tpu as pltpu
from jax.experimental.pallas import tpu_sc as plsc

N = 10000
E = 320000
D = 128
H = 128
C = 64

L = 16
NC = 2
NS = 16
NW = NC * NS
CH = 128
KMAX = 80
KPH = 16
EP = NW * KMAX * CH
NP = 10240
RPT = NP // NS
ZR = 32

_mesh = plsc.VectorSubcoreMesh(core_axis_name="c", subcore_axis_name="s")


def _make_spmm(Dg, Ds):
    in_place = Dg == Ds
    scratch = [
        pltpu.VMEM((KPH * 2, CH), jnp.int32),
        pltpu.VMEM((KPH, CH), jnp.float32),
        pltpu.VMEM((CH,), jnp.int32),
        pltpu.VMEM((CH,), jnp.int32),
        pltpu.VMEM((CH, Dg), jnp.float32),
        pltpu.VMEM((ZR, Ds), jnp.float32),
        pltpu.VMEM_SHARED((NP, Ds), jnp.float32),
        pltpu.SemaphoreType.DMA,
        pltpu.SemaphoreType.DMA,
    ]
    if not in_place:
        scratch.insert(5, pltpu.VMEM((CH, Ds), jnp.float32))

    @functools.partial(
        pl.kernel,
        out_type=(jax.ShapeDtypeStruct((NP, Ds), jnp.float32),
                  jax.ShapeDtypeStruct((NP, Ds), jnp.float32)),
        mesh=_mesh,
        scratch_types=scratch,
    )
    def spmm(m_hbm, idx_hbm, wts_hbm, out0_hbm, out1_hbm,
             idx2, wts, col_v, row_v, rows_a, *rest):
        if in_place:
            zbuf, acc, sem_a, sem_b = rest
        else:
            rows_s, zbuf, acc, sem_a, sem_b = rest
        c = lax.axis_index("c")
        s = lax.axis_index("s")
        wid = c * NS + s

        def fetch_tables(p, sem):
            pltpu.async_copy(
                idx_hbm.at[pl.ds(wid * KMAX * 2 + p * 2 * KPH, 2 * KPH)],
                idx2, sem)
            pltpu.async_copy(
                wts_hbm.at[pl.ds(wid * KMAX + p * KPH, KPH)], wts, sem)

        def wait_tables(p, sem):
            pltpu.make_async_copy(
                idx_hbm.at[pl.ds(wid * KMAX * 2 + p * 2 * KPH, 2 * KPH)],
                idx2, sem).wait()
            pltpu.make_async_copy(
                wts_hbm.at[pl.ds(wid * KMAX + p * KPH, KPH)],
                wts, sem).wait()

        fetch_tables(0, sem_a)

        zvec = jnp.zeros((L,), jnp.float32)

        def zrow(i, carry):
            for j in range(Ds // L):
                zbuf[i, pl.ds(j * L, L)] = zvec
            return carry

        lax.fori_loop(0, ZR, zrow, 0)
        for k in range(RPT // ZR):
            pltpu.sync_copy(zbuf, acc.at[pl.ds(s * RPT + k * ZR, ZR)])
        wait_tables(0, sem_a)
        plsc.subcore_barrier()

        def process(lk, rows):
            for g in range(CH // L):
                col_v[pl.ds(g * L, L)] = idx2[2 * lk, pl.ds(g * L, L)]
                row_v[pl.ds(g * L, L)] = idx2[2 * lk + 1, pl.ds(g * L, L)]
            pltpu.async_copy(m_hbm.at[col_v], rows, sem_b).wait()
            dst = rows if in_place else rows_s

            def edge_group(g, icarry):
                wvec = wts[lk, pl.ds(g * L, L)]
                for t in range(L):
                    wgt = wvec[t]
                    i = g * L + t
                    for j in range(Ds // L):
                        dst[i, pl.ds(j * L, L)] = (
                            rows[i, pl.ds(j * L, L)] * wgt)
                return icarry

            lax.fori_loop(0, CH // L, edge_group, 0)
            pltpu.sync_copy(dst, acc.at[row_v], add=True)

        for p in range(KMAX // KPH):
            if p > 0:
                fetch_tables(p, sem_a)
                wait_tables(p, sem_a)

            def chunk_body(lk, carry):
                process(lk, rows_a)
                return carry

            lax.fori_loop(0, KPH, chunk_body, 0)
        plsc.subcore_barrier()

        r0 = s * RPT

        @pl.when(c == 0)
        def _():
            pltpu.sync_copy(acc.at[pl.ds(r0, RPT)], out0_hbm.at[pl.ds(r0, RPT)])

        @pl.when(c == 1)
        def _():
            pltpu.sync_copy(acc.at[pl.ds(r0, RPT)], out1_hbm.at[pl.ds(r0, RPT)])

    return spmm


_spmm1 = _make_spmm(D, D)

_MM_BLK = 2000


def _mm1(x, w1):
    def body(x_ref, w_ref, o_ref):
        o_ref[...] = jnp.dot(x_ref[...], w_ref[...],
                             preferred_element_type=jnp.float32)

    return pl.pallas_call(
        body,
        grid=(N // _MM_BLK,),
        in_specs=[pl.BlockSpec((_MM_BLK, D), lambda i: (i, 0)),
                  pl.BlockSpec((D, H), lambda i: (0, 0))],
        out_specs=pl.BlockSpec((_MM_BLK, H), lambda i: (i, 0)),
        out_shape=jax.ShapeDtypeStruct((N, H), jnp.float32),
    )(x, w1)


def _relu_add_mm2(s0, s1, w2p):
    def body(a_ref, b_ref, w_ref, o_ref):
        h = jnp.maximum(a_ref[...] + b_ref[...], 0.0)
        o_ref[...] = jnp.dot(h, w_ref[...], preferred_element_type=jnp.float32)

    return pl.pallas_call(
        body,
        grid=(N // _MM_BLK,),
        in_specs=[pl.BlockSpec((_MM_BLK, H), lambda i: (i, 0)),
                  pl.BlockSpec((_MM_BLK, H), lambda i: (i, 0)),
                  pl.BlockSpec((H, D), lambda i: (0, 0))],
        out_specs=pl.BlockSpec((_MM_BLK, D), lambda i: (i, 0)),
        out_shape=jax.ShapeDtypeStruct((N, D), jnp.float32),
    )(s0, s1, w2p)


def _add_partials(t0, t1):
    def body(a_ref, b_ref, o_ref):
        o_ref[...] = a_ref[:, :C] + b_ref[:, :C]

    return pl.pallas_call(
        body,
        grid=(N // _MM_BLK,),
        in_specs=[pl.BlockSpec((_MM_BLK, D), lambda i: (i, 0)),
                  pl.BlockSpec((_MM_BLK, D), lambda i: (i, 0))],
        out_specs=pl.BlockSpec((_MM_BLK, C), lambda i: (i, 0)),
        out_shape=jax.ShapeDtypeStruct((N, C), jnp.float32),
    )(t0, t1)


@jax.jit
def _run(x, idx, wts, w1, w2p):
    p = _mm1(x, w1)
    s0, s1 = _spmm1(p, idx, wts)
    q = _relu_add_mm2(s0, s1, w2p)
    t0, t1 = _spmm1(q, idx, wts)
    return _add_partials(t0, t1)


def kernel(x, edge_index, edge_weight, W1, W2):
    pad = EP - E
    col = jnp.concatenate([edge_index[1],
                           jnp.zeros((pad,), edge_index.dtype)])
    row = jnp.concatenate([edge_index[0],
                           jnp.full((pad,), NP - 1, edge_index.dtype)])
    idx = jnp.stack([col.reshape(NW, KMAX, CH).astype(jnp.int32),
                     row.reshape(NW, KMAX, CH).astype(jnp.int32)],
                    axis=2).reshape(NW * KMAX * 2, CH)
    wts = jnp.concatenate(
        [edge_weight, jnp.zeros((pad,), edge_weight.dtype)]).reshape(
            NW * KMAX, CH)
    w2p = jnp.pad(W2, ((0, 0), (0, D - C)))
    return _run(x, idx, wts, W1, w2p)
